# baseline (speedup 1.0000x reference)
import functools

import jax
import jax.numpy as jnp
from jax.experimental import pallas as pl
from jax.experimental.pallas import tpu as pltpu


def _body(ls_ref, h_ref, probs_ref, out_ref,
          accr_ref, acci_ref, psum_ref, hrbf_ref, hibf_ref,
          prbf_ref, pibf_ref, psq_ref):
    p = pl.program_id(0)
    i = pl.program_id(1)
    nb = pl.num_programs(1)
    nblk = out_ref.shape[0]

    @pl.when(p == 0)
    def _phase_protos():
        pb = probs_ref[...]
        hr = h_ref[:, 0, :].astype(jnp.bfloat16)
        hi = h_ref[:, 1, :].astype(jnp.bfloat16)
        hrbf_ref[pl.ds(i * nblk, nblk), :] = hr
        hibf_ref[pl.ds(i * nblk, nblk), :] = hi
        pbf = pb.astype(jnp.bfloat16)
        part_r = jax.lax.dot_general(
            pbf, hr, (((0,), (0,)), ((), ())),
            preferred_element_type=jnp.float32)
        part_i = jax.lax.dot_general(
            pbf, hi, (((0,), (0,)), ((), ())),
            preferred_element_type=jnp.float32)
        ssum = jnp.sum(pb, axis=0)[None, :]

        @pl.when(i == 0)
        def _():
            accr_ref[...] = part_r
            acci_ref[...] = part_i
            psum_ref[...] = ssum

        @pl.when(i > 0)
        def _():
            accr_ref[...] += part_r
            acci_ref[...] += part_i
            psum_ref[...] += ssum

        @pl.when(i == nb - 1)
        def _():
            cnt = psum_ref[0, :]
            cnt = jnp.where(cnt == 0.0, 1.0, cnt)
            pr = accr_ref[...] / cnt[:, None]
            pi_ = acci_ref[...] / cnt[:, None]
            prbf_ref[...] = pr.astype(jnp.bfloat16)
            pibf_ref[...] = pi_.astype(jnp.bfloat16)
            psq_ref[...] = (jnp.sum(pr * pr, axis=1)
                            + jnp.sum(pi_ * pi_, axis=1))[None, :]

    @pl.when(p == 1)
    def _phase_logits():
        hr = hrbf_ref[pl.ds(i * nblk, nblk), :]
        hi = hibf_ref[pl.ds(i * nblk, nblk), :]
        cross = jax.lax.dot_general(
            hr, prbf_ref[...], (((1,), (1,)), ((), ())),
            preferred_element_type=jnp.float32)
        cross += jax.lax.dot_general(
            hi, pibf_ref[...], (((1,), (1,)), ((), ())),
            preferred_element_type=jnp.float32)
        hrf = hr.astype(jnp.float32)
        hif = hi.astype(jnp.float32)
        h_sq = (jnp.sum(hrf * hrf, axis=1)
                + jnp.sum(hif * hif, axis=1))[:, None]
        scale = -0.5 * jnp.exp(-ls_ref[0])
        out_ref[...] = (h_sq - 2.0 * cross + psq_ref[...]) * scale


@functools.partial(jax.jit, static_argnames=("interpret",))
def _run(h, probs, log_sigma_l, interpret=False):
    B, N, two, D = h.shape
    K = probs.shape[-1]
    h3 = h.reshape(N, two, D)
    pz = probs.reshape(N, K)

    nb = 8
    nblk = N // nb
    out = pl.pallas_call(
        _body,
        grid=(2, nb),
        in_specs=[
            pl.BlockSpec(memory_space=pltpu.SMEM),
            pl.BlockSpec((nblk, two, D),
                         lambda p, i: (jnp.where(p == 0, i, nb - 1), 0, 0)),
            pl.BlockSpec((nblk, K),
                         lambda p, i: (jnp.where(p == 0, i, nb - 1), 0)),
        ],
        out_specs=pl.BlockSpec((nblk, K),
                               lambda p, i: (jnp.where(p == 0, 0, i), 0)),
        out_shape=jax.ShapeDtypeStruct((N, K), jnp.float32),
        scratch_shapes=[
            pltpu.VMEM((K, D), jnp.float32),
            pltpu.VMEM((K, D), jnp.float32),
            pltpu.VMEM((1, K), jnp.float32),
            pltpu.VMEM((N, D), jnp.bfloat16),
            pltpu.VMEM((N, D), jnp.bfloat16),
            pltpu.VMEM((K, D), jnp.bfloat16),
            pltpu.VMEM((K, D), jnp.bfloat16),
            pltpu.VMEM((1, K), jnp.float32),
        ],
        interpret=interpret,
    )(log_sigma_l, h3, pz)

    return out.reshape(B, N, K)


def kernel(h, probs, log_sigma_l):
    return _run(h, probs, log_sigma_l)


# speedup vs baseline: 1.8232x; 1.8232x over previous
import functools

import jax
import jax.numpy as jnp
from jax.experimental import pallas as pl
from jax.experimental.pallas import tpu as pltpu


def _body(ls_ref, hc_ref, probs_ref, out_ref,
          acc_ref, psum_ref, pbf_ref, psq_ref):
    p = pl.program_id(0)
    i = pl.program_id(1)
    nb = pl.num_programs(1)
    nblk = out_ref.shape[0]

    @pl.when(p == 0)
    def _phase_protos():
        pb = probs_ref[...]
        hbf = hc_ref[pl.ds(i * nblk, nblk), :]
        part = jax.lax.dot_general(
            pb.astype(jnp.bfloat16), hbf,
            (((0,), (0,)), ((), ())),
            preferred_element_type=jnp.float32)
        ssum = jnp.sum(pb, axis=0)[None, :]

        @pl.when(i == 0)
        def _():
            acc_ref[...] = part
            psum_ref[...] = ssum

        @pl.when(i > 0)
        def _():
            acc_ref[...] += part
            psum_ref[...] += ssum

        @pl.when(i == nb - 1)
        def _():
            cnt = psum_ref[0, :]
            cnt = jnp.where(cnt == 0.0, 1.0, cnt)
            pr = acc_ref[...] / cnt[:, None]
            pbf_ref[...] = pr.astype(jnp.bfloat16)
            psq_ref[...] = jnp.sum(pr * pr, axis=1)[None, :]

    @pl.when(p == 1)
    def _phase_logits():
        hbf = hc_ref[pl.ds(i * nblk, nblk), :]
        cross = jax.lax.dot_general(
            hbf, pbf_ref[...],
            (((1,), (1,)), ((), ())),
            preferred_element_type=jnp.float32)
        hf = hbf.astype(jnp.float32)
        h_sq = jnp.sum(hf * hf, axis=1, keepdims=True)
        scale = -0.5 * jnp.exp(-ls_ref[0])
        out_ref[...] = (h_sq - 2.0 * cross + psq_ref[...]) * scale


@functools.partial(jax.jit, static_argnames=("interpret",))
def _run(h, probs, log_sigma_l, interpret=False):
    B, N, two, D = h.shape
    K = probs.shape[-1]
    D2 = two * D
    hcb = h.reshape(N, D2).astype(jnp.bfloat16)
    pz = probs.reshape(N, K)

    nb = 4
    nblk = N // nb
    out = pl.pallas_call(
        _body,
        grid=(2, nb),
        in_specs=[
            pl.BlockSpec(memory_space=pltpu.SMEM),
            pl.BlockSpec((N, D2), lambda p, i: (0, 0)),
            pl.BlockSpec((nblk, K),
                         lambda p, i: (jnp.where(p == 0, i, nb - 1), 0)),
        ],
        out_specs=pl.BlockSpec((nblk, K),
                               lambda p, i: (jnp.where(p == 0, 0, i), 0)),
        out_shape=jax.ShapeDtypeStruct((N, K), jnp.float32),
        scratch_shapes=[
            pltpu.VMEM((K, D2), jnp.float32),
            pltpu.VMEM((1, K), jnp.float32),
            pltpu.VMEM((K, D2), jnp.bfloat16),
            pltpu.VMEM((1, K), jnp.float32),
        ],
        interpret=interpret,
    )(log_sigma_l, hcb, pz)

    return out.reshape(B, N, K)


def kernel(h, probs, log_sigma_l):
    return _run(h, probs, log_sigma_l)
